# baseline (speedup 1.0000x reference)
import jax
import jax.numpy as jnp
from jax.experimental import pallas as pl
from jax.experimental.pallas import tpu as pltpu


def _gate_body(c_ref, t_ref, wt_ref, bfc_ref, w1t_ref, bfc1_ref,
               m_ref, w_ref):
    cb = c_ref[...]
    tb = t_ref[...]
    wt = wt_ref[...]
    h = (jnp.dot(wt[:, :64], cb, preferred_element_type=jnp.float32)
         + jnp.dot(wt[:, 64:], tb, preferred_element_type=jnp.float32)
         + bfc_ref[...])
    h = jnp.where(h > 0, h, jnp.exp(h) - 1.0)
    h = jnp.dot(w1t_ref[...], h, preferred_element_type=jnp.float32) \
        + bfc1_ref[...]
    w = jax.nn.sigmoid(h)
    w_ref[...] = w
    m_ref[...] = tb + (cb - tb) * w


def kernel(c, t, W_fc, b_fc, W_fc1, b_fc1):
    bs, n, dim = c.shape
    ct = c.transpose(0, 2, 1)
    tt = t.transpose(0, 2, 1)
    WT = W_fc.T
    W1T = W_fc1.T
    bfc = b_fc.reshape(dim, 1)
    bfc1 = b_fc1.reshape(dim, 1)

    BN = 10240
    nb = -(-n // BN)
    grid = (bs, nb)

    spec = pl.BlockSpec((None, dim, BN), lambda b, i: (b, 0, i))
    rep = lambda shape: pl.BlockSpec(shape, lambda b, i: (0, 0))

    mt, wt_out = pl.pallas_call(
        _gate_body,
        grid=grid,
        in_specs=[
            spec, spec,
            rep((dim, 2 * dim)),
            rep((dim, 1)),
            rep((dim, dim)),
            rep((dim, 1)),
        ],
        out_specs=[spec, spec],
        out_shape=[
            jax.ShapeDtypeStruct((bs, dim, n), jnp.float32),
            jax.ShapeDtypeStruct((bs, dim, n), jnp.float32),
        ],
        compiler_params=pltpu.CompilerParams(
            dimension_semantics=("parallel", "parallel"),
        ),
    )(ct, tt, WT, bfc, W1T, bfc1)

    return mt.transpose(0, 2, 1), wt_out.transpose(0, 2, 1)


# speedup vs baseline: 1.0531x; 1.0531x over previous
import jax
import jax.numpy as jnp
from jax.experimental import pallas as pl
from jax.experimental.pallas import tpu as pltpu


def _gate_body(c_ref, t_ref, wt_ref, bfc_ref, w1t_ref, bfc1_ref,
               m_ref, w_ref):
    cb = c_ref[...]
    tb = t_ref[...]
    wt = wt_ref[...]
    h = (jnp.dot(wt[:, :64], cb, preferred_element_type=jnp.float32)
         + jnp.dot(wt[:, 64:], tb, preferred_element_type=jnp.float32)
         + bfc_ref[...])
    h = jnp.where(h > 0, h, jnp.exp(h) - 1.0)
    h = jnp.dot(w1t_ref[...], h, preferred_element_type=jnp.float32) \
        + bfc1_ref[...]
    w = jax.nn.sigmoid(h)
    w_ref[...] = w
    m_ref[...] = tb + (cb - tb) * w


def kernel(c, t, W_fc, b_fc, W_fc1, b_fc1):
    bs, n, dim = c.shape
    ct = c.transpose(0, 2, 1)
    tt = t.transpose(0, 2, 1)
    WT = W_fc.T
    W1T = W_fc1.T
    bfc = b_fc.reshape(dim, 1)
    bfc1 = b_fc1.reshape(dim, 1)

    BN = 20480
    nb = -(-n // BN)
    grid = (bs, nb)

    spec = pl.BlockSpec((None, dim, BN), lambda b, i: (b, 0, i))
    rep = lambda shape: pl.BlockSpec(shape, lambda b, i: (0, 0))

    mt, wt_out = pl.pallas_call(
        _gate_body,
        grid=grid,
        in_specs=[
            spec, spec,
            rep((dim, 2 * dim)),
            rep((dim, 1)),
            rep((dim, dim)),
            rep((dim, 1)),
        ],
        out_specs=[spec, spec],
        out_shape=[
            jax.ShapeDtypeStruct((bs, dim, n), jnp.float32),
            jax.ShapeDtypeStruct((bs, dim, n), jnp.float32),
        ],
        compiler_params=pltpu.CompilerParams(
            dimension_semantics=("parallel", "parallel"),
        ),
    )(ct, tt, WT, bfc, W1T, bfc1)

    return mt.transpose(0, 2, 1), wt_out.transpose(0, 2, 1)
